# baseline (speedup 1.0000x reference)
import functools

import jax
import jax.numpy as jnp
from jax.experimental import pallas as pl
import jax.experimental.pallas.tpu as pltpu


def _vq_body(zt_ref, cb_ref, o_ref, mval_ref, midx_ref, *, bk, n_kc):
    kc = pl.program_id(1)

    @pl.when(kc == 0)
    def _init():
        mval_ref[...] = jnp.full(mval_ref.shape, jnp.inf, jnp.float32)
        midx_ref[...] = jnp.zeros(midx_ref.shape, jnp.int32)

    lat = zt_ref[0]
    zz = jnp.sum(lat * lat, axis=0, keepdims=True)
    cb = cb_ref[...]
    e2 = jnp.sum(cb * cb, axis=1, keepdims=True)
    s = jax.lax.dot_general(
        cb, lat, (((1,), (0,)), ((), ())),
        preferred_element_type=jnp.float32)
    d = (zz + e2) - 2.0 * s

    cmin = jnp.min(d, axis=0, keepdims=True)
    rows = jax.lax.broadcasted_iota(jnp.int32, d.shape, 0)
    cidx = jnp.min(jnp.where(d == cmin, rows, bk), axis=0, keepdims=True)
    cidx = cidx + kc * bk

    upd = cmin < mval_ref[...]
    midx_ref[...] = jnp.where(upd, cidx, midx_ref[...])
    mval_ref[...] = jnp.where(upd, cmin, mval_ref[...])

    @pl.when(kc == n_kc - 1)
    def _emit():
        o_ref[0] = midx_ref[...]


def kernel(latent, codebook):
    B, C, H, W = latent.shape
    K, _ = codebook.shape
    n_tok = H * W
    zt = latent.reshape(B, C, n_tok)

    BK = 1024
    n_kc = K // BK

    out = pl.pallas_call(
        functools.partial(_vq_body, bk=BK, n_kc=n_kc),
        grid=(B, n_kc),
        in_specs=[
            pl.BlockSpec((1, C, n_tok), lambda b, kc: (b, 0, 0)),
            pl.BlockSpec((BK, C), lambda b, kc: (kc, 0)),
        ],
        out_specs=pl.BlockSpec((1, 1, n_tok), lambda b, kc: (b, 0, 0)),
        out_shape=jax.ShapeDtypeStruct((B, 1, n_tok), jnp.int32),
        scratch_shapes=[
            pltpu.VMEM((1, n_tok), jnp.float32),
            pltpu.VMEM((1, n_tok), jnp.int32),
        ],
        compiler_params=pltpu.CompilerParams(
            dimension_semantics=("parallel", "arbitrary"),
        ),
    )(zt, codebook)
    return out.reshape(B, n_tok)


# speedup vs baseline: 2.0770x; 2.0770x over previous
import functools

import jax
import jax.numpy as jnp
from jax.experimental import pallas as pl
import jax.experimental.pallas.tpu as pltpu


def _vq_body(zt_ref, cb_ref, o_ref, *, bk, n_kc):
    lat = zt_ref[0]
    zz = jnp.sum(lat * lat, axis=0, keepdims=True)

    w = lat.shape[1]
    mval = jnp.full((1, w), jnp.inf, jnp.float32)
    midx = jnp.zeros((1, w), jnp.float32)

    for kc in range(n_kc):
        cb = cb_ref[kc * bk:(kc + 1) * bk, :]
        e2 = jnp.sum(cb * cb, axis=1, keepdims=True)
        s = jax.lax.dot_general(
            cb, lat, (((1,), (0,)), ((), ())),
            preferred_element_type=jnp.float32)
        d = (zz + e2) - 2.0 * s

        cmin = jnp.min(d, axis=0, keepdims=True)
        rows = jax.lax.broadcasted_iota(jnp.int32, d.shape, 0).astype(jnp.float32)
        cidx = jnp.min(jnp.where(d == cmin, rows, jnp.inf),
                       axis=0, keepdims=True) + float(kc * bk)

        upd = cmin < mval
        midx = jnp.where(upd, cidx, midx)
        mval = jnp.where(upd, cmin, mval)

    o_ref[0] = midx.astype(jnp.int32)


def kernel(latent, codebook):
    B, C, H, W = latent.shape
    K, _ = codebook.shape
    n_tok = H * W
    zt = latent.reshape(B, C, n_tok)

    BK = 1024
    n_kc = K // BK

    out = pl.pallas_call(
        functools.partial(_vq_body, bk=BK, n_kc=n_kc),
        grid=(B,),
        in_specs=[
            pl.BlockSpec((1, C, n_tok), lambda b: (b, 0, 0)),
            pl.BlockSpec((K, C), lambda b: (0, 0)),
        ],
        out_specs=pl.BlockSpec((1, 1, n_tok), lambda b: (b, 0, 0)),
        out_shape=jax.ShapeDtypeStruct((B, 1, n_tok), jnp.int32),
        compiler_params=pltpu.CompilerParams(
            dimension_semantics=("arbitrary",),
        ),
    )(zt, codebook)
    return out.reshape(B, n_tok)


# speedup vs baseline: 2.9077x; 1.4000x over previous
import functools

import jax
import jax.numpy as jnp
from jax.experimental import pallas as pl
import jax.experimental.pallas.tpu as pltpu


def _vq_body(zt_ref, cb_ref, o_ref, cbm2_ref, e2_ref, *, bk, n_kc, sl_rows):
    b = pl.program_id(0)
    k_total, c_dim = cb_ref.shape
    w = zt_ref.shape[2]

    @pl.when(b == 0)
    def _build():
        for kc in range(n_kc):
            rows = slice(kc * bk, (kc + 1) * bk)
            cb = cb_ref[rows, :]
            cbm2_ref[rows, :] = cb * -2.0
            e2 = jnp.sum(cb * cb, axis=1, keepdims=True)
            e2_ref[rows, :] = jnp.broadcast_to(e2, (bk, w))

    lat = zt_ref[0]
    zz = jnp.sum(lat * lat, axis=0, keepdims=True)

    n_slices = bk // sl_rows
    rm = jnp.full((sl_rows, w), jnp.inf, jnp.float32)
    si = jnp.zeros((sl_rows, w), jnp.float32)

    for kc in range(n_kc):
        sm2 = jax.lax.dot_general(
            cbm2_ref[kc * bk:(kc + 1) * bk, :], lat,
            (((1,), (0,)), ((), ())),
            preferred_element_type=jnp.float32)
        for sl in range(n_slices):
            rows = slice(sl * sl_rows, (sl + 1) * sl_rows)
            t = zz + e2_ref[kc * bk + sl * sl_rows:
                            kc * bk + (sl + 1) * sl_rows, :]
            d = t + sm2[rows, :]
            gs = jnp.float32(kc * n_slices + sl)
            upd = d < rm
            si = jnp.where(upd, gs, si)
            rm = jnp.where(upd, d, rm)

    pos = jax.lax.broadcasted_iota(jnp.int32, (sl_rows, w), 0).astype(jnp.float32)
    rows_g = si * jnp.float32(sl_rows) + pos
    gmin = jnp.min(rm, axis=0, keepdims=True)
    cand = jnp.where(rm == gmin, rows_g, jnp.inf)
    best = jnp.min(cand, axis=0, keepdims=True)
    o_ref[0] = best.astype(jnp.int32)


def kernel(latent, codebook):
    B, C, H, W = latent.shape
    K, _ = codebook.shape
    n_tok = H * W
    zt = latent.reshape(B, C, n_tok)

    BK = 1024
    n_kc = K // BK
    SL = 32

    out = pl.pallas_call(
        functools.partial(_vq_body, bk=BK, n_kc=n_kc, sl_rows=SL),
        grid=(B,),
        in_specs=[
            pl.BlockSpec((1, C, n_tok), lambda b: (b, 0, 0)),
            pl.BlockSpec((K, C), lambda b: (0, 0)),
        ],
        out_specs=pl.BlockSpec((1, 1, n_tok), lambda b: (b, 0, 0)),
        out_shape=jax.ShapeDtypeStruct((B, 1, n_tok), jnp.int32),
        scratch_shapes=[
            pltpu.VMEM((K, C), jnp.float32),
            pltpu.VMEM((K, n_tok), jnp.float32),
        ],
        compiler_params=pltpu.CompilerParams(
            dimension_semantics=("arbitrary",),
        ),
    )(zt, codebook)
    return out.reshape(B, n_tok)
